# baseline (speedup 1.0000x reference)
import functools

import jax
import jax.numpy as jnp
from jax import lax
from jax.experimental import pallas as pl
from jax.experimental.pallas import tpu as pltpu
from jax.experimental.pallas import tpu_sc as plsc

N = 10000
NP = 10240
E = 320000
F = 128
K = 8

NC = 2
NS = 16
NW = NC * NS
EW = 125
ER = E // EW
RPW = ER // NW
RPT = NP // NS

_B = 400
HALF = NP // NC
TPC = HALF // NS
RPT_D = ER // NS


@functools.cache
def _sc_mesh():
    return plsc.VectorSubcoreMesh(
        core_axis_name="c", subcore_axis_name="s",
        num_cores=NC, num_subcores=NS)


@functools.cache
def _pre_call():
    return pl.kernel(
        _pre_body,
        out_type=[
            jax.ShapeDtypeStruct((N, F), jnp.float32),
            jax.ShapeDtypeStruct((N, F), jnp.float32),
            jax.ShapeDtypeStruct((N,), jnp.float32),
        ],
        mesh=_sc_mesh(),
        scratch_types=[
            pltpu.VMEM((RPT_D, EW), jnp.int32),
            pltpu.VMEM((128,), jnp.float32),
            pltpu.VMEM((RPT,), jnp.float32),
            pltpu.VMEM((TPC,), jnp.float32),
            pltpu.VMEM((80, F), jnp.float32),
            pltpu.VMEM((80, F), jnp.float32),
            pltpu.SemaphoreType.DMA,
            pltpu.VMEM_SHARED((NP,), jnp.float32),
        ],
    )


def _pre_body(dst_hbm, x_hbm, y_hbm, xs_hbm, dis_hbm,
              idx_v, ones_v, zero_v, disv, xbuf, ybuf, dsem, deg_sh):
    c = lax.axis_index("c")
    s = lax.axis_index("s")

    def _zero(i, _):
        zero_v[pl.ds(i * 16, 16)] = jnp.zeros((16,), jnp.float32)
        return 0

    lax.fori_loop(0, RPT // 16, _zero, 0)
    for j in range(8):
        ones_v[pl.ds(j * 16, 16)] = jnp.ones((16,), jnp.float32)
    pltpu.sync_copy(zero_v, deg_sh.at[pl.ds(s * RPT, RPT)])
    plsc.subcore_barrier()

    pltpu.sync_copy(dst_hbm.at[pl.ds(s * RPT_D, RPT_D)], idx_v)

    def _scat(t, _):
        pltpu.async_copy(ones_v.at[pl.ds(0, EW)],
                         deg_sh.at[idx_v.at[t]], dsem, add=True)

        @pl.when(t >= 8)
        def _():
            pltpu.make_async_copy(ones_v.at[pl.ds(0, EW)],
                                  deg_sh.at[idx_v.at[0]], dsem).wait()

        return 0

    lax.fori_loop(0, RPT_D, _scat, 0)

    def _drain(t, _):
        pltpu.make_async_copy(ones_v.at[pl.ds(0, EW)],
                              deg_sh.at[idx_v.at[0]], dsem).wait()
        return 0

    lax.fori_loop(0, 8, _drain, 0)
    plsc.subcore_barrier()

    base = c * HALF + s * TPC
    pltpu.sync_copy(deg_sh.at[pl.ds(base, TPC)], disv)

    def _newton(i, _):
        d = disv[pl.ds(i * 16, 16)] + 1.0
        r = 1.0 / d
        for _i in range(24):
            r = r * (1.5 - 0.5 * d * r * r)
        disv[pl.ds(i * 16, 16)] = r
        return 0

    lax.fori_loop(0, TPC // 16, _newton, 0)

    nch = jnp.minimum(TPC // 80, (N - base) // 80)

    def _chunk(j, _):
        row0 = base + j * 80
        pltpu.sync_copy(x_hbm.at[pl.ds(row0, 80)], xbuf)

        def _grp(g, _2):
            dvec = disv[pl.ds(j * 80 + g * 16, 16)]
            for r16 in range(16):
                r = g * 16 + r16
                dsc = dvec.at[jnp.full((16,), r16, jnp.int32)].get(
                    mode="promise_in_bounds")
                for l in range(F // 16):
                    sl = pl.ds(l * 16, 16)
                    yv = xbuf[r, sl] * dsc
                    ybuf[r, sl] = yv
                    xbuf[r, sl] = yv * dsc
            return 0

        lax.fori_loop(0, 80 // 16, _grp, 0)
        pltpu.sync_copy(ybuf, y_hbm.at[pl.ds(row0, 80)])
        pltpu.sync_copy(xbuf, xs_hbm.at[pl.ds(row0, 80)])
        pltpu.sync_copy(disv.at[pl.ds(j * 80, 80)],
                        dis_hbm.at[pl.ds(row0, 80)])
        return 0

    lax.fori_loop(0, nch, _chunk, 0)


_CH = 40


@functools.cache
def _agg_call():
    return pl.kernel(
        _agg_body,
        out_type=jax.ShapeDtypeStruct((NC, NP, F), jnp.float32),
        mesh=_sc_mesh(),
        scratch_types=[
            pltpu.VMEM((_CH, EW), jnp.int32),
            pltpu.VMEM((_CH, EW), jnp.int32),
            pltpu.VMEM((EW, F), jnp.float32),
            pltpu.VMEM((EW, F), jnp.float32),
            pltpu.SemaphoreType.DMA,
            pltpu.SemaphoreType.DMA,
            pltpu.SemaphoreType.DMA,
            pltpu.SemaphoreType.DMA,
            pltpu.VMEM_SHARED((NP, F), jnp.float32),
        ],
    )


def _agg_body(src_hbm, dst_hbm, y_hbm, zp_hbm,
              idx_s, idx_d, buf0, buf1, g0, g1, s0, s1, z_sh):
    c = lax.axis_index("c")
    s = lax.axis_index("s")
    w = c * NS + s

    bufs = (buf0, buf1)
    gsem = (g0, g1)
    ssem = (s0, s1)

    def fire_g(t, b):
        pltpu.async_copy(y_hbm.at[idx_s.at[t]], bufs[b], gsem[b])

    def wait_g(b):
        pltpu.make_async_copy(y_hbm.at[idx_s.at[0]], bufs[b], gsem[b]).wait()

    def fire_s(t, b):
        pltpu.async_copy(bufs[b], z_sh.at[idx_d.at[t]], ssem[b], add=True)

    def wait_s(b):
        pltpu.make_async_copy(bufs[b], z_sh.at[idx_d.at[0]], ssem[b]).wait()

    def _zero(r, _):
        for j in range(F // 16):
            buf0[r, pl.ds(j * 16, 16)] = jnp.zeros((16,), jnp.float32)
        return 0

    lax.fori_loop(0, EW, _zero, 0)
    for j in range(RPT // 80):
        pltpu.sync_copy(buf0.at[pl.ds(0, 80)],
                        z_sh.at[pl.ds(s * RPT + j * 80, 80)])
    plsc.subcore_barrier()

    for chunk in range(RPW // _CH):
        pltpu.sync_copy(src_hbm.at[pl.ds(w * RPW + chunk * _CH, _CH)], idx_s)
        pltpu.sync_copy(dst_hbm.at[pl.ds(w * RPW + chunk * _CH, _CH)], idx_d)

        fire_g(0, 0)
        wait_g(0)
        fire_g(1, 1)
        fire_s(0, 0)
        wait_g(1)
        wait_s(0)
        fire_g(2, 0)
        fire_s(1, 1)

        def _pair(tp, _):
            t0 = 2 * tp
            wait_g(0)
            wait_s(1)
            fire_g(t0 + 1, 1)
            fire_s(t0, 0)
            wait_g(1)
            wait_s(0)

            @pl.when(tp != _CH // 2 - 1)
            def _():
                fire_g(t0 + 2, 0)

            fire_s(t0 + 1, 1)
            return 0

        lax.fori_loop(1, _CH // 2, _pair, 0)
        wait_s(1)

    plsc.subcore_barrier()
    pltpu.sync_copy(z_sh.at[pl.ds(s * RPT, RPT)],
                    zp_hbm.at[c, pl.ds(s * RPT, RPT)])


def _mm_body(zp_ref, dis_ref, xs_ref, Ws_ref, bs_ref, out_ref):
    z = zp_ref[0] + zp_ref[1]
    zs = dis_ref[...] * z + xs_ref[...]
    for k in range(K):
        hk = lax.dot_general(zs, Ws_ref[k], (((1,), (1,)), ((), ())),
                             preferred_element_type=jnp.float32)
        out_ref[:, k, :] = hk + bs_ref[k]


_mm_call = pl.pallas_call(
    _mm_body,
    grid=(N // _B,),
    in_specs=[
        pl.BlockSpec((NC, _B, F), lambda i: (0, i, 0)),
        pl.BlockSpec((_B, 1), lambda i: (i, 0)),
        pl.BlockSpec((_B, F), lambda i: (i, 0)),
        pl.BlockSpec((K, F, F), lambda i: (0, 0, 0)),
        pl.BlockSpec((K, 1, F), lambda i: (0, 0, 0)),
    ],
    out_specs=pl.BlockSpec((_B, K, F), lambda i: (i, 0, 0)),
    out_shape=jax.ShapeDtypeStruct((N, K, F), jnp.float32),
)


def kernel(x, edge_index, Ws, bs):
    src2d = edge_index[0].reshape(ER, EW)
    dst2d = edge_index[1].reshape(ER, EW)
    y, xs, dis = _pre_call()(dst2d, x)
    zp = _agg_call()(src2d, dst2d, y)
    return _mm_call(zp, dis.reshape(N, 1), xs, Ws, bs.reshape(K, 1, F))


# speedup vs baseline: 1.1032x; 1.1032x over previous
import functools

import jax
import jax.numpy as jnp
from jax import lax
from jax.experimental import pallas as pl
from jax.experimental.pallas import tpu as pltpu
from jax.experimental.pallas import tpu_sc as plsc

N = 10000
NP = 10240
E = 320000
F = 128
K = 8

NC = 2
NS = 16
NW = NC * NS
EW = 125
ER = E // EW
RPW = ER // NW
RPT = NP // NS

_B = 400
HALF = NP // NC
TPC = HALF // NS
RPT_D = ER // NS


@functools.cache
def _sc_mesh():
    return plsc.VectorSubcoreMesh(
        core_axis_name="c", subcore_axis_name="s",
        num_cores=NC, num_subcores=NS)


@functools.cache
def _pre_call():
    return pl.kernel(
        _pre_body,
        out_type=[
            jax.ShapeDtypeStruct((N, F), jnp.float32),
            jax.ShapeDtypeStruct((N, F), jnp.float32),
            jax.ShapeDtypeStruct((N,), jnp.float32),
        ],
        mesh=_sc_mesh(),
        scratch_types=[
            pltpu.VMEM((RPT_D, EW), jnp.int32),
            pltpu.VMEM((128,), jnp.float32),
            pltpu.VMEM((RPT,), jnp.float32),
            pltpu.VMEM((TPC,), jnp.float32),
            pltpu.VMEM((80, F), jnp.float32),
            pltpu.VMEM((80, F), jnp.float32),
            pltpu.SemaphoreType.DMA,
            pltpu.VMEM_SHARED((NP,), jnp.float32),
        ],
    )


def _pre_body(dst_hbm, x_hbm, y_hbm, xs_hbm, dis_hbm,
              idx_v, ones_v, zero_v, disv, xbuf, ybuf, dsem, deg_sh):
    c = lax.axis_index("c")
    s = lax.axis_index("s")

    def _zero(i, _):
        zero_v[pl.ds(i * 16, 16)] = jnp.zeros((16,), jnp.float32)
        return 0

    lax.fori_loop(0, RPT // 16, _zero, 0)
    for j in range(8):
        ones_v[pl.ds(j * 16, 16)] = jnp.ones((16,), jnp.float32)
    pltpu.sync_copy(zero_v, deg_sh.at[pl.ds(s * RPT, RPT)])
    plsc.subcore_barrier()

    pltpu.sync_copy(dst_hbm.at[pl.ds(s * RPT_D, RPT_D)], idx_v)

    def _scat(t, _):
        pltpu.async_copy(ones_v.at[pl.ds(0, EW)],
                         deg_sh.at[idx_v.at[t]], dsem, add=True)

        @pl.when(t >= 8)
        def _():
            pltpu.make_async_copy(ones_v.at[pl.ds(0, EW)],
                                  deg_sh.at[idx_v.at[0]], dsem).wait()

        return 0

    lax.fori_loop(0, RPT_D, _scat, 0)

    def _drain(t, _):
        pltpu.make_async_copy(ones_v.at[pl.ds(0, EW)],
                              deg_sh.at[idx_v.at[0]], dsem).wait()
        return 0

    lax.fori_loop(0, 8, _drain, 0)
    plsc.subcore_barrier()

    base = c * HALF + s * TPC
    pltpu.sync_copy(deg_sh.at[pl.ds(base, TPC)], disv)

    def _newton(i, _):
        d = disv[pl.ds(i * 16, 16)] + 1.0
        r = 1.0 / d
        for _i in range(24):
            r = r * (1.5 - 0.5 * d * r * r)
        disv[pl.ds(i * 16, 16)] = r
        return 0

    lax.fori_loop(0, TPC // 16, _newton, 0)

    nch = jnp.minimum(TPC // 80, (N - base) // 80)

    def _chunk(j, _):
        row0 = base + j * 80
        pltpu.sync_copy(x_hbm.at[pl.ds(row0, 80)], xbuf)

        def _grp(g, _2):
            dvec = disv[pl.ds(j * 80 + g * 16, 16)]
            for r16 in range(16):
                r = g * 16 + r16
                dsc = dvec.at[jnp.full((16,), r16, jnp.int32)].get(
                    mode="promise_in_bounds")
                for l in range(F // 16):
                    sl = pl.ds(l * 16, 16)
                    yv = xbuf[r, sl] * dsc
                    ybuf[r, sl] = yv
                    xbuf[r, sl] = yv * dsc
            return 0

        lax.fori_loop(0, 80 // 16, _grp, 0)
        pltpu.sync_copy(ybuf, y_hbm.at[pl.ds(row0, 80)])
        pltpu.sync_copy(xbuf, xs_hbm.at[pl.ds(row0, 80)])
        pltpu.sync_copy(disv.at[pl.ds(j * 80, 80)],
                        dis_hbm.at[pl.ds(row0, 80)])
        return 0

    lax.fori_loop(0, nch, _chunk, 0)


_CH = 40


@functools.cache
def _agg_call():
    return pl.kernel(
        _agg_body,
        out_type=jax.ShapeDtypeStruct((NC, NP, F), jnp.float32),
        mesh=_sc_mesh(),
        scratch_types=[
            pltpu.VMEM((_CH, EW), jnp.int32),
            pltpu.VMEM((_CH, EW), jnp.int32),
            pltpu.VMEM((EW, F), jnp.float32),
            pltpu.VMEM((EW, F), jnp.float32),
            pltpu.SemaphoreType.DMA,
            pltpu.SemaphoreType.DMA,
            pltpu.SemaphoreType.DMA,
            pltpu.SemaphoreType.DMA,
            pltpu.VMEM_SHARED((NP, F), jnp.float32),
        ],
    )


def _agg_body(src_hbm, dst_hbm, y_hbm, zp_hbm,
              idx_s, idx_d, buf0, buf1, g0, g1, s0, s1, z_sh):
    c = lax.axis_index("c")
    s = lax.axis_index("s")
    w = c * NS + s

    bufs = (buf0, buf1)
    gsem = (g0, g1)
    ssem = (s0, s1)

    def fire_g(t, b):
        pltpu.async_copy(y_hbm.at[idx_s.at[t]], bufs[b], gsem[b])

    def wait_g(b):
        pltpu.make_async_copy(y_hbm.at[idx_s.at[0]], bufs[b], gsem[b]).wait()

    def fire_s(t, b):
        pltpu.async_copy(bufs[b], z_sh.at[idx_d.at[t]], ssem[b], add=True)

    def wait_s(b):
        pltpu.make_async_copy(bufs[b], z_sh.at[idx_d.at[0]], ssem[b]).wait()

    def _zero(r, _):
        for j in range(F // 16):
            buf0[r, pl.ds(j * 16, 16)] = jnp.zeros((16,), jnp.float32)
        return 0

    lax.fori_loop(0, EW, _zero, 0)
    for j in range(RPT // 80):
        pltpu.sync_copy(buf0.at[pl.ds(0, 80)],
                        z_sh.at[pl.ds(s * RPT + j * 80, 80)])
    plsc.subcore_barrier()

    for chunk in range(RPW // _CH):
        pltpu.sync_copy(src_hbm.at[pl.ds(w * RPW + chunk * _CH, _CH)], idx_s)
        pltpu.sync_copy(dst_hbm.at[pl.ds(w * RPW + chunk * _CH, _CH)], idx_d)

        fire_g(0, 0)

        def _pair(tp, _):
            t0 = 2 * tp
            fire_g(t0 + 1, 1)
            wait_g(0)
            pltpu.sync_copy(buf0, z_sh.at[idx_d.at[t0]], add=True)

            @pl.when(tp != _CH // 2 - 1)
            def _():
                fire_g(t0 + 2, 0)

            wait_g(1)
            pltpu.sync_copy(buf1, z_sh.at[idx_d.at[t0 + 1]], add=True)
            return 0

        lax.fori_loop(0, _CH // 2, _pair, 0)

    plsc.subcore_barrier()
    pltpu.sync_copy(z_sh.at[pl.ds(s * RPT, RPT)],
                    zp_hbm.at[c, pl.ds(s * RPT, RPT)])


def _mm_body(zp_ref, dis_ref, xs_ref, Ws_ref, bs_ref, out_ref):
    z = zp_ref[0] + zp_ref[1]
    zs = dis_ref[...] * z + xs_ref[...]
    for k in range(K):
        hk = lax.dot_general(zs, Ws_ref[k], (((1,), (1,)), ((), ())),
                             preferred_element_type=jnp.float32)
        out_ref[:, k, :] = hk + bs_ref[k]


_mm_call = pl.pallas_call(
    _mm_body,
    grid=(N // _B,),
    in_specs=[
        pl.BlockSpec((NC, _B, F), lambda i: (0, i, 0)),
        pl.BlockSpec((_B, 1), lambda i: (i, 0)),
        pl.BlockSpec((_B, F), lambda i: (i, 0)),
        pl.BlockSpec((K, F, F), lambda i: (0, 0, 0)),
        pl.BlockSpec((K, 1, F), lambda i: (0, 0, 0)),
    ],
    out_specs=pl.BlockSpec((_B, K, F), lambda i: (i, 0, 0)),
    out_shape=jax.ShapeDtypeStruct((N, K, F), jnp.float32),
)


def kernel(x, edge_index, Ws, bs):
    src2d = edge_index[0].reshape(ER, EW)
    dst2d = edge_index[1].reshape(ER, EW)
    y, xs, dis = _pre_call()(dst2d, x)
    zp = _agg_call()(src2d, dst2d, y)
    return _mm_call(zp, dis.reshape(N, 1), xs, Ws, bs.reshape(K, 1, F))


# speedup vs baseline: 1.1055x; 1.0021x over previous
import functools

import jax
import jax.numpy as jnp
from jax import lax
from jax.experimental import pallas as pl
from jax.experimental.pallas import tpu as pltpu
from jax.experimental.pallas import tpu_sc as plsc

N = 10000
NP = 10240
E = 320000
F = 128
K = 8

NC = 2
NS = 16
NW = NC * NS
EW = 128
ER = 2560
EPAD = ER * EW - E
RPW = ER // NW
RPT = NP // NS

_B = 400
HALF = NP // NC
TPC = HALF // NS
RPT_D = ER // NS


@functools.cache
def _sc_mesh():
    return plsc.VectorSubcoreMesh(
        core_axis_name="c", subcore_axis_name="s",
        num_cores=NC, num_subcores=NS)


@functools.cache
def _pre_call():
    return pl.kernel(
        _pre_body,
        out_type=[
            jax.ShapeDtypeStruct((N, F), jnp.float32),
            jax.ShapeDtypeStruct((N, F), jnp.float32),
            jax.ShapeDtypeStruct((N,), jnp.float32),
        ],
        mesh=_sc_mesh(),
        scratch_types=[
            pltpu.VMEM((RPT_D, EW), jnp.int32),
            pltpu.VMEM((128,), jnp.float32),
            pltpu.VMEM((RPT,), jnp.float32),
            pltpu.VMEM((TPC,), jnp.float32),
            pltpu.VMEM((80, F), jnp.float32),
            pltpu.VMEM((80, F), jnp.float32),
            pltpu.SemaphoreType.DMA,
            pltpu.VMEM_SHARED((NP,), jnp.float32),
        ],
    )


def _pre_body(dst_hbm, x_hbm, y_hbm, xs_hbm, dis_hbm,
              idx_v, ones_v, zero_v, disv, xbuf, ybuf, dsem, deg_sh):
    c = lax.axis_index("c")
    s = lax.axis_index("s")

    def _zero(i, _):
        zero_v[pl.ds(i * 16, 16)] = jnp.zeros((16,), jnp.float32)
        return 0

    lax.fori_loop(0, RPT // 16, _zero, 0)
    for j in range(8):
        ones_v[pl.ds(j * 16, 16)] = jnp.ones((16,), jnp.float32)
    pltpu.sync_copy(zero_v, deg_sh.at[pl.ds(s * RPT, RPT)])
    plsc.subcore_barrier()

    pltpu.sync_copy(dst_hbm.at[pl.ds(s * RPT_D, RPT_D)], idx_v)

    def _scat(t, _):
        pltpu.async_copy(ones_v.at[pl.ds(0, EW)],
                         deg_sh.at[idx_v.at[t]], dsem, add=True)

        @pl.when(t >= 8)
        def _():
            pltpu.make_async_copy(ones_v.at[pl.ds(0, EW)],
                                  deg_sh.at[idx_v.at[0]], dsem).wait()

        return 0

    lax.fori_loop(0, RPT_D, _scat, 0)

    def _drain(t, _):
        pltpu.make_async_copy(ones_v.at[pl.ds(0, EW)],
                              deg_sh.at[idx_v.at[0]], dsem).wait()
        return 0

    lax.fori_loop(0, 8, _drain, 0)
    plsc.subcore_barrier()

    base = c * HALF + s * TPC
    pltpu.sync_copy(deg_sh.at[pl.ds(base, TPC)], disv)

    def _newton(i, _):
        d = disv[pl.ds(i * 16, 16)] + 1.0
        r = 1.0 / d
        for _i in range(24):
            r = r * (1.5 - 0.5 * d * r * r)
        disv[pl.ds(i * 16, 16)] = r
        return 0

    lax.fori_loop(0, TPC // 16, _newton, 0)

    nch = jnp.minimum(TPC // 80, (N - base) // 80)

    def _chunk(j, _):
        row0 = base + j * 80
        pltpu.sync_copy(x_hbm.at[pl.ds(row0, 80)], xbuf)

        def _grp(g, _2):
            dvec = disv[pl.ds(j * 80 + g * 16, 16)]
            for r16 in range(16):
                r = g * 16 + r16
                dsc = dvec.at[jnp.full((16,), r16, jnp.int32)].get(
                    mode="promise_in_bounds")
                for l in range(F // 16):
                    sl = pl.ds(l * 16, 16)
                    yv = xbuf[r, sl] * dsc
                    ybuf[r, sl] = yv
                    xbuf[r, sl] = yv * dsc
            return 0

        lax.fori_loop(0, 80 // 16, _grp, 0)
        pltpu.sync_copy(ybuf, y_hbm.at[pl.ds(row0, 80)])
        pltpu.sync_copy(xbuf, xs_hbm.at[pl.ds(row0, 80)])
        pltpu.sync_copy(disv.at[pl.ds(j * 80, 80)],
                        dis_hbm.at[pl.ds(row0, 80)])
        return 0

    lax.fori_loop(0, nch, _chunk, 0)


_CH = 40


@functools.cache
def _agg_call():
    return pl.kernel(
        _agg_body,
        out_type=jax.ShapeDtypeStruct((NC, NP, F), jnp.float32),
        mesh=_sc_mesh(),
        scratch_types=[
            pltpu.VMEM((_CH, EW), jnp.int32),
            pltpu.VMEM((_CH, EW), jnp.int32),
            pltpu.VMEM((EW, F), jnp.float32),
            pltpu.VMEM((EW, F), jnp.float32),
            pltpu.SemaphoreType.DMA,
            pltpu.SemaphoreType.DMA,
            pltpu.SemaphoreType.DMA,
            pltpu.SemaphoreType.DMA,
            pltpu.VMEM_SHARED((NP, F), jnp.float32),
        ],
    )


def _agg_body(src_hbm, dst_hbm, y_hbm, zp_hbm,
              idx_s, idx_d, buf0, buf1, g0, g1, s0, s1, z_sh):
    c = lax.axis_index("c")
    s = lax.axis_index("s")
    w = c * NS + s

    bufs = (buf0, buf1)
    gsem = (g0, g1)
    ssem = (s0, s1)

    def fire_g(t, b):
        pltpu.async_copy(y_hbm.at[idx_s.at[t]], bufs[b], gsem[b])

    def wait_g(b):
        pltpu.make_async_copy(y_hbm.at[idx_s.at[0]], bufs[b], gsem[b]).wait()

    def fire_s(t, b):
        pltpu.async_copy(bufs[b], z_sh.at[idx_d.at[t]], ssem[b], add=True)

    def wait_s(b):
        pltpu.make_async_copy(bufs[b], z_sh.at[idx_d.at[0]], ssem[b]).wait()

    def _zero(r, _):
        for j in range(F // 16):
            buf0[r, pl.ds(j * 16, 16)] = jnp.zeros((16,), jnp.float32)
        return 0

    lax.fori_loop(0, EW, _zero, 0)
    for j in range(RPT // 80):
        pltpu.sync_copy(buf0.at[pl.ds(0, 80)],
                        z_sh.at[pl.ds(s * RPT + j * 80, 80)])
    plsc.subcore_barrier()

    for chunk in range(RPW // _CH):
        pltpu.sync_copy(src_hbm.at[pl.ds(w * RPW + chunk * _CH, _CH)], idx_s)
        pltpu.sync_copy(dst_hbm.at[pl.ds(w * RPW + chunk * _CH, _CH)], idx_d)

        fire_g(0, 0)

        def _pair(tp, _):
            t0 = 2 * tp
            fire_g(t0 + 1, 1)
            wait_g(0)
            pltpu.sync_copy(buf0, z_sh.at[idx_d.at[t0]], add=True)

            @pl.when(tp != _CH // 2 - 1)
            def _():
                fire_g(t0 + 2, 0)

            wait_g(1)
            pltpu.sync_copy(buf1, z_sh.at[idx_d.at[t0 + 1]], add=True)
            return 0

        lax.fori_loop(0, _CH // 2, _pair, 0)

    plsc.subcore_barrier()
    pltpu.sync_copy(z_sh.at[pl.ds(s * RPT, RPT)],
                    zp_hbm.at[c, pl.ds(s * RPT, RPT)])


def _mm_body(zp_ref, dis_ref, xs_ref, Ws_ref, bs_ref, out_ref):
    z = zp_ref[0] + zp_ref[1]
    zs = dis_ref[...] * z + xs_ref[...]
    for k in range(K):
        hk = lax.dot_general(zs, Ws_ref[k], (((1,), (1,)), ((), ())),
                             preferred_element_type=jnp.float32)
        out_ref[:, k, :] = hk + bs_ref[k]


_mm_call = pl.pallas_call(
    _mm_body,
    grid=(N // _B,),
    in_specs=[
        pl.BlockSpec((NC, _B, F), lambda i: (0, i, 0)),
        pl.BlockSpec((_B, 1), lambda i: (i, 0)),
        pl.BlockSpec((_B, F), lambda i: (i, 0)),
        pl.BlockSpec((K, F, F), lambda i: (0, 0, 0)),
        pl.BlockSpec((K, 1, F), lambda i: (0, 0, 0)),
    ],
    out_specs=pl.BlockSpec((_B, K, F), lambda i: (i, 0, 0)),
    out_shape=jax.ShapeDtypeStruct((N, K, F), jnp.float32),
)


def kernel(x, edge_index, Ws, bs):
    pad_src = jnp.arange(EPAD, dtype=jnp.int32) % N
    pad_dst = N + jnp.arange(EPAD, dtype=jnp.int32) % (NP - N)
    src2d = jnp.concatenate([edge_index[0], pad_src]).reshape(ER, EW)
    dst2d = jnp.concatenate([edge_index[1], pad_dst]).reshape(ER, EW)
    y, xs, dis = _pre_call()(dst2d, x)
    zp = _agg_call()(src2d, dst2d, y)
    return _mm_call(zp, dis.reshape(N, 1), xs, Ws, bs.reshape(K, 1, F))


# speedup vs baseline: 1.1420x; 1.0329x over previous
import functools

import jax
import jax.numpy as jnp
from jax import lax
from jax.experimental import pallas as pl
from jax.experimental.pallas import tpu as pltpu
from jax.experimental.pallas import tpu_sc as plsc

N = 10000
NP = 10240
E = 320000
F = 128
K = 8

NC = 2
NS = 16
NW = NC * NS
EW = 128
ER = 2560
EPAD = ER * EW - E
RPW = ER // NW
RPT = NP // NS

_B = 400
HALF = NP // NC
TPC = HALF // NS
RPT_D = ER // NS


@functools.cache
def _sc_mesh():
    return plsc.VectorSubcoreMesh(
        core_axis_name="c", subcore_axis_name="s",
        num_cores=NC, num_subcores=NS)


@functools.cache
def _pre_call():
    return pl.kernel(
        _pre_body,
        out_type=[
            jax.ShapeDtypeStruct((N, F), jnp.float32),
            jax.ShapeDtypeStruct((N, F), jnp.float32),
            jax.ShapeDtypeStruct((N,), jnp.float32),
        ],
        mesh=_sc_mesh(),
        scratch_types=[
            pltpu.VMEM((RPT_D, EW), jnp.int32),
            pltpu.VMEM((128,), jnp.float32),
            pltpu.VMEM((RPT,), jnp.float32),
            pltpu.VMEM((TPC,), jnp.float32),
            pltpu.VMEM((80, F), jnp.float32),
            pltpu.VMEM((80, F), jnp.float32),
            pltpu.SemaphoreType.DMA,
            pltpu.VMEM_SHARED((NP,), jnp.float32),
        ],
    )


def _pre_body(ei_hbm, x_hbm, y_hbm, xs_hbm, dis_hbm,
              idx_v, ones_v, zero_v, disv, xbuf, ybuf, dsem, deg_sh):
    c = lax.axis_index("c")
    s = lax.axis_index("s")

    def _zero(i, _):
        zero_v[pl.ds(i * 16, 16)] = jnp.zeros((16,), jnp.float32)
        return 0

    lax.fori_loop(0, RPT // 16, _zero, 0)
    for j in range(8):
        ones_v[pl.ds(j * 16, 16)] = jnp.ones((16,), jnp.float32)
    pltpu.sync_copy(zero_v, deg_sh.at[pl.ds(s * RPT, RPT)])
    plsc.subcore_barrier()

    pltpu.sync_copy(ei_hbm.at[1, pl.ds(s * RPT_D, RPT_D)], idx_v)

    def _scat(t, _):
        pltpu.async_copy(ones_v.at[pl.ds(0, EW)],
                         deg_sh.at[idx_v.at[t]], dsem, add=True)

        @pl.when(t >= 8)
        def _():
            pltpu.make_async_copy(ones_v.at[pl.ds(0, EW)],
                                  deg_sh.at[idx_v.at[0]], dsem).wait()

        return 0

    lax.fori_loop(0, RPT_D, _scat, 0)

    def _drain(t, _):
        pltpu.make_async_copy(ones_v.at[pl.ds(0, EW)],
                              deg_sh.at[idx_v.at[0]], dsem).wait()
        return 0

    lax.fori_loop(0, 8, _drain, 0)
    plsc.subcore_barrier()

    base = c * HALF + s * TPC
    pltpu.sync_copy(deg_sh.at[pl.ds(base, TPC)], disv)

    def _newton(i, _):
        d = disv[pl.ds(i * 16, 16)] + 1.0
        r = 1.0 / d
        for _i in range(24):
            r = r * (1.5 - 0.5 * d * r * r)
        disv[pl.ds(i * 16, 16)] = r
        return 0

    lax.fori_loop(0, TPC // 16, _newton, 0)

    nch = jnp.minimum(TPC // 80, (N - base) // 80)

    def _chunk(j, _):
        row0 = base + j * 80
        pltpu.sync_copy(x_hbm.at[pl.ds(row0, 80)], xbuf)

        def _grp(g, _2):
            dvec = disv[pl.ds(j * 80 + g * 16, 16)]
            for r16 in range(16):
                r = g * 16 + r16
                dsc = dvec.at[jnp.full((16,), r16, jnp.int32)].get(
                    mode="promise_in_bounds")
                for l in range(F // 16):
                    sl = pl.ds(l * 16, 16)
                    yv = xbuf[r, sl] * dsc
                    ybuf[r, sl] = yv
                    xbuf[r, sl] = yv * dsc
            return 0

        lax.fori_loop(0, 80 // 16, _grp, 0)
        pltpu.sync_copy(ybuf, y_hbm.at[pl.ds(row0, 80)])
        pltpu.sync_copy(xbuf, xs_hbm.at[pl.ds(row0, 80)])
        pltpu.sync_copy(disv.at[pl.ds(j * 80, 80)],
                        dis_hbm.at[pl.ds(row0, 80)])
        return 0

    lax.fori_loop(0, nch, _chunk, 0)


_CH = 40


@functools.cache
def _agg_call():
    return pl.kernel(
        _agg_body,
        out_type=jax.ShapeDtypeStruct((NC, NP, F), jnp.float32),
        mesh=_sc_mesh(),
        scratch_types=[
            pltpu.VMEM((_CH, EW), jnp.int32),
            pltpu.VMEM((_CH, EW), jnp.int32),
            pltpu.VMEM((EW, F), jnp.float32),
            pltpu.VMEM((EW, F), jnp.float32),
            pltpu.SemaphoreType.DMA,
            pltpu.SemaphoreType.DMA,
            pltpu.SemaphoreType.DMA,
            pltpu.SemaphoreType.DMA,
            pltpu.VMEM_SHARED((NP, F), jnp.float32),
        ],
    )


def _agg_body(ei_hbm, y_hbm, zp_hbm,
              idx_s, idx_d, buf0, buf1, g0, g1, s0, s1, z_sh):
    c = lax.axis_index("c")
    s = lax.axis_index("s")
    w = c * NS + s

    bufs = (buf0, buf1)
    gsem = (g0, g1)
    ssem = (s0, s1)

    def fire_g(t, b):
        pltpu.async_copy(y_hbm.at[idx_s.at[t]], bufs[b], gsem[b])

    def wait_g(b):
        pltpu.make_async_copy(y_hbm.at[idx_s.at[0]], bufs[b], gsem[b]).wait()

    def fire_s(t, b):
        pltpu.async_copy(bufs[b], z_sh.at[idx_d.at[t]], ssem[b], add=True)

    def wait_s(b):
        pltpu.make_async_copy(bufs[b], z_sh.at[idx_d.at[0]], ssem[b]).wait()

    def _zero(r, _):
        for j in range(F // 16):
            buf0[r, pl.ds(j * 16, 16)] = jnp.zeros((16,), jnp.float32)
        return 0

    lax.fori_loop(0, EW, _zero, 0)
    for j in range(RPT // 80):
        pltpu.sync_copy(buf0.at[pl.ds(0, 80)],
                        z_sh.at[pl.ds(s * RPT + j * 80, 80)])
    plsc.subcore_barrier()

    for chunk in range(RPW // _CH):
        pltpu.sync_copy(ei_hbm.at[0, pl.ds(w * RPW + chunk * _CH, _CH)], idx_s)
        pltpu.sync_copy(ei_hbm.at[1, pl.ds(w * RPW + chunk * _CH, _CH)], idx_d)

        fire_g(0, 0)

        def _pair(tp, _):
            t0 = 2 * tp
            fire_g(t0 + 1, 1)
            wait_g(0)
            pltpu.sync_copy(buf0, z_sh.at[idx_d.at[t0]], add=True)

            @pl.when(tp != _CH // 2 - 1)
            def _():
                fire_g(t0 + 2, 0)

            wait_g(1)
            pltpu.sync_copy(buf1, z_sh.at[idx_d.at[t0 + 1]], add=True)
            return 0

        lax.fori_loop(0, _CH // 2, _pair, 0)

    plsc.subcore_barrier()
    pltpu.sync_copy(z_sh.at[pl.ds(s * RPT, RPT)],
                    zp_hbm.at[c, pl.ds(s * RPT, RPT)])


def _mm_body(zp_ref, dis_ref, xs_ref, Ws_ref, bs_ref, out_ref):
    z = zp_ref[0] + zp_ref[1]
    zs = dis_ref[...] * z + xs_ref[...]
    for k in range(K):
        hk = lax.dot_general(zs, Ws_ref[k], (((1,), (1,)), ((), ())),
                             preferred_element_type=jnp.float32)
        out_ref[:, k, :] = hk + bs_ref[k]


_mm_call = pl.pallas_call(
    _mm_body,
    grid=(N // _B,),
    in_specs=[
        pl.BlockSpec((NC, _B, F), lambda i: (0, i, 0)),
        pl.BlockSpec((_B, 1), lambda i: (i, 0)),
        pl.BlockSpec((_B, F), lambda i: (i, 0)),
        pl.BlockSpec((K, F, F), lambda i: (0, 0, 0)),
        pl.BlockSpec((K, 1, F), lambda i: (0, 0, 0)),
    ],
    out_specs=pl.BlockSpec((_B, K, F), lambda i: (i, 0, 0)),
    out_shape=jax.ShapeDtypeStruct((N, K, F), jnp.float32),
)


def kernel(x, edge_index, Ws, bs):
    pad_src = jnp.arange(EPAD, dtype=jnp.int32) % N
    pad_dst = N + jnp.arange(EPAD, dtype=jnp.int32) % (NP - N)
    ei3 = jnp.concatenate(
        [edge_index, jnp.stack([pad_src, pad_dst])], axis=1).reshape(2, ER, EW)
    y, xs, dis = _pre_call()(ei3, x)
    zp = _agg_call()(ei3, y)
    return _mm_call(zp, dis.reshape(N, 1), xs, Ws, bs.reshape(K, 1, F))


# speedup vs baseline: 1.1873x; 1.0397x over previous
import functools

import jax
import jax.numpy as jnp
from jax import lax
from jax.experimental import pallas as pl
from jax.experimental.pallas import tpu as pltpu
from jax.experimental.pallas import tpu_sc as plsc

N = 10000
NP = 10240
E = 320000
F = 128
K = 8

NC = 2
NS = 16
NW = NC * NS
EW = 128
ER = 2560
EPAD = ER * EW - E
RPW = ER // NW
RPT = NP // NS

_B = 1000
HALF = NP // NC
TPC = HALF // NS
RPT_D = ER // NS


@functools.cache
def _sc_mesh():
    return plsc.VectorSubcoreMesh(
        core_axis_name="c", subcore_axis_name="s",
        num_cores=NC, num_subcores=NS)


@functools.cache
def _pre_call():
    return pl.kernel(
        _pre_body,
        out_type=[
            jax.ShapeDtypeStruct((N, F), jnp.float32),
            jax.ShapeDtypeStruct((N, F), jnp.float32),
            jax.ShapeDtypeStruct((N,), jnp.float32),
        ],
        mesh=_sc_mesh(),
        scratch_types=[
            pltpu.VMEM((RPT_D, EW), jnp.int32),
            pltpu.VMEM((128,), jnp.float32),
            pltpu.VMEM((RPT,), jnp.float32),
            pltpu.VMEM((TPC,), jnp.float32),
            pltpu.VMEM((80, F), jnp.float32),
            pltpu.VMEM((80, F), jnp.float32),
            pltpu.SemaphoreType.DMA,
            pltpu.VMEM_SHARED((NP,), jnp.float32),
        ],
    )


def _pre_body(ei_hbm, x_hbm, y_hbm, xs_hbm, dis_hbm,
              idx_v, ones_v, zero_v, disv, xbuf, ybuf, dsem, deg_sh):
    c = lax.axis_index("c")
    s = lax.axis_index("s")

    def _zero(i, _):
        zero_v[pl.ds(i * 16, 16)] = jnp.zeros((16,), jnp.float32)
        return 0

    lax.fori_loop(0, RPT // 16, _zero, 0)
    for j in range(8):
        ones_v[pl.ds(j * 16, 16)] = jnp.ones((16,), jnp.float32)
    pltpu.sync_copy(zero_v, deg_sh.at[pl.ds(s * RPT, RPT)])
    plsc.subcore_barrier()

    pltpu.sync_copy(ei_hbm.at[1, pl.ds(s * RPT_D, RPT_D)], idx_v)

    def _scat(t, _):
        pltpu.async_copy(ones_v.at[pl.ds(0, EW)],
                         deg_sh.at[idx_v.at[t]], dsem, add=True)

        @pl.when(t >= 8)
        def _():
            pltpu.make_async_copy(ones_v.at[pl.ds(0, EW)],
                                  deg_sh.at[idx_v.at[0]], dsem).wait()

        return 0

    lax.fori_loop(0, RPT_D, _scat, 0)

    def _drain(t, _):
        pltpu.make_async_copy(ones_v.at[pl.ds(0, EW)],
                              deg_sh.at[idx_v.at[0]], dsem).wait()
        return 0

    lax.fori_loop(0, 8, _drain, 0)
    plsc.subcore_barrier()

    base = c * HALF + s * TPC
    pltpu.sync_copy(deg_sh.at[pl.ds(base, TPC)], disv)

    def _newton(i, _):
        d = disv[pl.ds(i * 16, 16)] + 1.0
        r = 1.0 / d
        for _i in range(24):
            r = r * (1.5 - 0.5 * d * r * r)
        disv[pl.ds(i * 16, 16)] = r
        return 0

    lax.fori_loop(0, TPC // 16, _newton, 0)

    nch = jnp.minimum(TPC // 80, (N - base) // 80)

    def _chunk(j, _):
        row0 = base + j * 80
        pltpu.sync_copy(x_hbm.at[pl.ds(row0, 80)], xbuf)

        def _grp(g, _2):
            dvec = disv[pl.ds(j * 80 + g * 16, 16)]
            for r16 in range(16):
                r = g * 16 + r16
                dsc = dvec.at[jnp.full((16,), r16, jnp.int32)].get(
                    mode="promise_in_bounds")
                for l in range(F // 16):
                    sl = pl.ds(l * 16, 16)
                    yv = xbuf[r, sl] * dsc
                    ybuf[r, sl] = yv
                    xbuf[r, sl] = yv * dsc
            return 0

        lax.fori_loop(0, 80 // 16, _grp, 0)
        pltpu.sync_copy(ybuf, y_hbm.at[pl.ds(row0, 80)])
        pltpu.sync_copy(xbuf, xs_hbm.at[pl.ds(row0, 80)])
        pltpu.sync_copy(disv.at[pl.ds(j * 80, 80)],
                        dis_hbm.at[pl.ds(row0, 80)])
        return 0

    lax.fori_loop(0, nch, _chunk, 0)


_CH = 40


@functools.cache
def _agg_call():
    return pl.kernel(
        _agg_body,
        out_type=jax.ShapeDtypeStruct((NC, NP, F), jnp.float32),
        mesh=_sc_mesh(),
        scratch_types=[
            pltpu.VMEM((_CH, EW), jnp.int32),
            pltpu.VMEM((_CH, EW), jnp.int32),
            pltpu.VMEM((EW, F), jnp.float32),
            pltpu.VMEM((EW, F), jnp.float32),
            pltpu.SemaphoreType.DMA,
            pltpu.SemaphoreType.DMA,
            pltpu.SemaphoreType.DMA,
            pltpu.SemaphoreType.DMA,
            pltpu.VMEM_SHARED((NP, F), jnp.float32),
        ],
    )


def _agg_body(ei_hbm, y_hbm, zp_hbm,
              idx_s, idx_d, buf0, buf1, g0, g1, s0, s1, z_sh):
    c = lax.axis_index("c")
    s = lax.axis_index("s")
    w = c * NS + s

    bufs = (buf0, buf1)
    gsem = (g0, g1)
    ssem = (s0, s1)

    def fire_g(t, b):
        pltpu.async_copy(y_hbm.at[idx_s.at[t]], bufs[b], gsem[b])

    def wait_g(b):
        pltpu.make_async_copy(y_hbm.at[idx_s.at[0]], bufs[b], gsem[b]).wait()

    def fire_s(t, b):
        pltpu.async_copy(bufs[b], z_sh.at[idx_d.at[t]], ssem[b], add=True)

    def wait_s(b):
        pltpu.make_async_copy(bufs[b], z_sh.at[idx_d.at[0]], ssem[b]).wait()

    def _zero(r, _):
        for j in range(F // 16):
            buf0[r, pl.ds(j * 16, 16)] = jnp.zeros((16,), jnp.float32)
        return 0

    lax.fori_loop(0, EW, _zero, 0)
    for j in range(RPT // 80):
        pltpu.sync_copy(buf0.at[pl.ds(0, 80)],
                        z_sh.at[pl.ds(s * RPT + j * 80, 80)])
    plsc.subcore_barrier()

    for chunk in range(RPW // _CH):
        pltpu.sync_copy(ei_hbm.at[0, pl.ds(w * RPW + chunk * _CH, _CH)], idx_s)
        pltpu.sync_copy(ei_hbm.at[1, pl.ds(w * RPW + chunk * _CH, _CH)], idx_d)

        fire_g(0, 0)

        def _pair(tp, _):
            t0 = 2 * tp
            fire_g(t0 + 1, 1)
            wait_g(0)
            pltpu.sync_copy(buf0, z_sh.at[idx_d.at[t0]], add=True)

            @pl.when(tp != _CH // 2 - 1)
            def _():
                fire_g(t0 + 2, 0)

            wait_g(1)
            pltpu.sync_copy(buf1, z_sh.at[idx_d.at[t0 + 1]], add=True)
            return 0

        lax.fori_loop(0, _CH // 2, _pair, 0)

    plsc.subcore_barrier()
    pltpu.sync_copy(z_sh.at[pl.ds(s * RPT, RPT)],
                    zp_hbm.at[c, pl.ds(s * RPT, RPT)])


def _mm_body(zp_ref, dis_ref, xs_ref, Ws_ref, bs_ref, out_ref):
    z = zp_ref[0] + zp_ref[1]
    zs = dis_ref[...] * z + xs_ref[...]
    for k in range(K):
        hk = lax.dot_general(zs, Ws_ref[k], (((1,), (1,)), ((), ())),
                             preferred_element_type=jnp.float32)
        out_ref[:, k, :] = hk + bs_ref[k]


_mm_call = pl.pallas_call(
    _mm_body,
    grid=(N // _B,),
    in_specs=[
        pl.BlockSpec((NC, _B, F), lambda i: (0, i, 0)),
        pl.BlockSpec((_B, 1), lambda i: (i, 0)),
        pl.BlockSpec((_B, F), lambda i: (i, 0)),
        pl.BlockSpec((K, F, F), lambda i: (0, 0, 0)),
        pl.BlockSpec((K, 1, F), lambda i: (0, 0, 0)),
    ],
    out_specs=pl.BlockSpec((_B, K, F), lambda i: (i, 0, 0)),
    out_shape=jax.ShapeDtypeStruct((N, K, F), jnp.float32),
)


def kernel(x, edge_index, Ws, bs):
    pad_src = jnp.arange(EPAD, dtype=jnp.int32) % N
    pad_dst = N + jnp.arange(EPAD, dtype=jnp.int32) % (NP - N)
    ei3 = jnp.concatenate(
        [edge_index, jnp.stack([pad_src, pad_dst])], axis=1).reshape(2, ER, EW)
    y, xs, dis = _pre_call()(ei3, x)
    zp = _agg_call()(ei3, y)
    return _mm_call(zp, dis.reshape(N, 1), xs, Ws, bs.reshape(K, 1, F))


# speedup vs baseline: 1.1914x; 1.0035x over previous
import functools

import jax
import jax.numpy as jnp
from jax import lax
from jax.experimental import pallas as pl
from jax.experimental.pallas import tpu as pltpu
from jax.experimental.pallas import tpu_sc as plsc

N = 10000
NP = 10240
E = 320000
F = 128
K = 8

NC = 2
NS = 16
NW = NC * NS
EW = 128
ER = 2560
EPAD = ER * EW - E
RPW = ER // NW
RPT = NP // NS

_B = 2000
HALF = NP // NC
TPC = HALF // NS
RPT_D = ER // NS


@functools.cache
def _sc_mesh():
    return plsc.VectorSubcoreMesh(
        core_axis_name="c", subcore_axis_name="s",
        num_cores=NC, num_subcores=NS)


@functools.cache
def _pre_call():
    return pl.kernel(
        _pre_body,
        out_type=[
            jax.ShapeDtypeStruct((N, F), jnp.float32),
            jax.ShapeDtypeStruct((N, F), jnp.float32),
            jax.ShapeDtypeStruct((N,), jnp.float32),
        ],
        mesh=_sc_mesh(),
        scratch_types=[
            pltpu.VMEM((RPT_D, EW), jnp.int32),
            pltpu.VMEM((128,), jnp.float32),
            pltpu.VMEM((RPT,), jnp.float32),
            pltpu.VMEM((TPC,), jnp.float32),
            pltpu.VMEM((80, F), jnp.float32),
            pltpu.VMEM((80, F), jnp.float32),
            pltpu.SemaphoreType.DMA,
            pltpu.VMEM_SHARED((NP,), jnp.float32),
        ],
    )


def _pre_body(ei_hbm, x_hbm, y_hbm, xs_hbm, dis_hbm,
              idx_v, ones_v, zero_v, disv, xbuf, ybuf, dsem, deg_sh):
    c = lax.axis_index("c")
    s = lax.axis_index("s")

    def _zero(i, _):
        zero_v[pl.ds(i * 16, 16)] = jnp.zeros((16,), jnp.float32)
        return 0

    lax.fori_loop(0, RPT // 16, _zero, 0)
    for j in range(8):
        ones_v[pl.ds(j * 16, 16)] = jnp.ones((16,), jnp.float32)
    pltpu.sync_copy(zero_v, deg_sh.at[pl.ds(s * RPT, RPT)])
    plsc.subcore_barrier()

    pltpu.sync_copy(ei_hbm.at[1, pl.ds(s * RPT_D, RPT_D)], idx_v)

    def _scat(t, _):
        pltpu.async_copy(ones_v.at[pl.ds(0, EW)],
                         deg_sh.at[idx_v.at[t]], dsem, add=True)

        @pl.when(t >= 8)
        def _():
            pltpu.make_async_copy(ones_v.at[pl.ds(0, EW)],
                                  deg_sh.at[idx_v.at[0]], dsem).wait()

        return 0

    lax.fori_loop(0, RPT_D, _scat, 0)

    def _drain(t, _):
        pltpu.make_async_copy(ones_v.at[pl.ds(0, EW)],
                              deg_sh.at[idx_v.at[0]], dsem).wait()
        return 0

    lax.fori_loop(0, 8, _drain, 0)
    plsc.subcore_barrier()

    base = c * HALF + s * TPC
    pltpu.sync_copy(deg_sh.at[pl.ds(base, TPC)], disv)

    def _newton(i, _):
        d = disv[pl.ds(i * 16, 16)] + 1.0
        r = 1.0 / d
        for _i in range(24):
            r = r * (1.5 - 0.5 * d * r * r)
        disv[pl.ds(i * 16, 16)] = r
        return 0

    lax.fori_loop(0, TPC // 16, _newton, 0)

    nch = jnp.minimum(TPC // 80, (N - base) // 80)

    def _chunk(j, _):
        row0 = base + j * 80
        pltpu.sync_copy(x_hbm.at[pl.ds(row0, 80)], xbuf)

        def _grp(g, _2):
            dvec = disv[pl.ds(j * 80 + g * 16, 16)]
            for r16 in range(16):
                r = g * 16 + r16
                dsc = dvec.at[jnp.full((16,), r16, jnp.int32)].get(
                    mode="promise_in_bounds")
                for l in range(F // 16):
                    sl = pl.ds(l * 16, 16)
                    yv = xbuf[r, sl] * dsc
                    ybuf[r, sl] = yv
                    xbuf[r, sl] = yv * dsc
            return 0

        lax.fori_loop(0, 80 // 16, _grp, 0)
        pltpu.sync_copy(ybuf, y_hbm.at[pl.ds(row0, 80)])
        pltpu.sync_copy(xbuf, xs_hbm.at[pl.ds(row0, 80)])
        pltpu.sync_copy(disv.at[pl.ds(j * 80, 80)],
                        dis_hbm.at[pl.ds(row0, 80)])
        return 0

    lax.fori_loop(0, nch, _chunk, 0)


_CH = 40


@functools.cache
def _agg_call():
    return pl.kernel(
        _agg_body,
        out_type=jax.ShapeDtypeStruct((NC, NP, F), jnp.float32),
        mesh=_sc_mesh(),
        scratch_types=[
            pltpu.VMEM((_CH, EW), jnp.int32),
            pltpu.VMEM((_CH, EW), jnp.int32),
            pltpu.VMEM((EW, F), jnp.float32),
            pltpu.VMEM((EW, F), jnp.float32),
            pltpu.SemaphoreType.DMA,
            pltpu.SemaphoreType.DMA,
            pltpu.SemaphoreType.DMA,
            pltpu.SemaphoreType.DMA,
            pltpu.VMEM_SHARED((NP, F), jnp.float32),
        ],
    )


def _agg_body(ei_hbm, y_hbm, zp_hbm,
              idx_s, idx_d, buf0, buf1, g0, g1, s0, s1, z_sh):
    c = lax.axis_index("c")
    s = lax.axis_index("s")
    w = c * NS + s

    bufs = (buf0, buf1)
    gsem = (g0, g1)
    ssem = (s0, s1)

    def fire_g(t, b):
        pltpu.async_copy(y_hbm.at[idx_s.at[t]], bufs[b], gsem[b])

    def wait_g(b):
        pltpu.make_async_copy(y_hbm.at[idx_s.at[0]], bufs[b], gsem[b]).wait()

    def fire_s(t, b):
        pltpu.async_copy(bufs[b], z_sh.at[idx_d.at[t]], ssem[b], add=True)

    def wait_s(b):
        pltpu.make_async_copy(bufs[b], z_sh.at[idx_d.at[0]], ssem[b]).wait()

    def _zero(r, _):
        for j in range(F // 16):
            buf0[r, pl.ds(j * 16, 16)] = jnp.zeros((16,), jnp.float32)
        return 0

    lax.fori_loop(0, EW, _zero, 0)
    for j in range(RPT // 80):
        pltpu.sync_copy(buf0.at[pl.ds(0, 80)],
                        z_sh.at[pl.ds(s * RPT + j * 80, 80)])
    plsc.subcore_barrier()

    for chunk in range(RPW // _CH):
        pltpu.sync_copy(ei_hbm.at[0, pl.ds(w * RPW + chunk * _CH, _CH)], idx_s)
        pltpu.sync_copy(ei_hbm.at[1, pl.ds(w * RPW + chunk * _CH, _CH)], idx_d)

        fire_g(0, 0)

        def _pair(tp, _):
            t0 = 2 * tp
            fire_g(t0 + 1, 1)
            wait_g(0)
            pltpu.sync_copy(buf0, z_sh.at[idx_d.at[t0]], add=True)

            @pl.when(tp != _CH // 2 - 1)
            def _():
                fire_g(t0 + 2, 0)

            wait_g(1)
            pltpu.sync_copy(buf1, z_sh.at[idx_d.at[t0 + 1]], add=True)
            return 0

        lax.fori_loop(0, _CH // 2, _pair, 0)

    plsc.subcore_barrier()
    pltpu.sync_copy(z_sh.at[pl.ds(s * RPT, RPT)],
                    zp_hbm.at[c, pl.ds(s * RPT, RPT)])


def _mm_body(zp_ref, dis_ref, xs_ref, Ws_ref, bs_ref, out_ref):
    z = zp_ref[0] + zp_ref[1]
    zs = dis_ref[...] * z + xs_ref[...]
    for k in range(K):
        hk = lax.dot_general(zs, Ws_ref[k], (((1,), (1,)), ((), ())),
                             preferred_element_type=jnp.float32)
        out_ref[:, k, :] = hk + bs_ref[k]


_mm_call = pl.pallas_call(
    _mm_body,
    grid=(N // _B,),
    in_specs=[
        pl.BlockSpec((NC, _B, F), lambda i: (0, i, 0)),
        pl.BlockSpec((_B, 1), lambda i: (i, 0)),
        pl.BlockSpec((_B, F), lambda i: (i, 0)),
        pl.BlockSpec((K, F, F), lambda i: (0, 0, 0)),
        pl.BlockSpec((K, 1, F), lambda i: (0, 0, 0)),
    ],
    out_specs=pl.BlockSpec((_B, K, F), lambda i: (i, 0, 0)),
    out_shape=jax.ShapeDtypeStruct((N, K, F), jnp.float32),
)


def kernel(x, edge_index, Ws, bs):
    pad_src = jnp.arange(EPAD, dtype=jnp.int32) % N
    pad_dst = N + jnp.arange(EPAD, dtype=jnp.int32) % (NP - N)
    ei3 = jnp.concatenate(
        [edge_index, jnp.stack([pad_src, pad_dst])], axis=1).reshape(2, ER, EW)
    y, xs, dis = _pre_call()(ei3, x)
    zp = _agg_call()(ei3, y)
    return _mm_call(zp, dis.reshape(N, 1), xs, Ws, bs.reshape(K, 1, F))


# speedup vs baseline: 1.2100x; 1.0156x over previous
import functools

import jax
import jax.numpy as jnp
from jax import lax
from jax.experimental import pallas as pl
from jax.experimental.pallas import tpu as pltpu
from jax.experimental.pallas import tpu_sc as plsc

N = 10000
NP = 10240
E = 320000
F = 128
K = 8

NC = 2
NS = 16
NW = NC * NS
EW = 128
ER = 2560
EPAD = ER * EW - E
RPW = ER // NW
RPT = NP // NS

_B = 2000
HALF = NP // NC
TPC = HALF // NS
RPT_D = ER // NS


@functools.cache
def _sc_mesh():
    return plsc.VectorSubcoreMesh(
        core_axis_name="c", subcore_axis_name="s",
        num_cores=NC, num_subcores=NS)


@functools.cache
def _pre_call():
    return pl.kernel(
        _pre_body,
        out_type=[
            jax.ShapeDtypeStruct((N, F), jnp.float32),
            jax.ShapeDtypeStruct((N, F), jnp.float32),
            jax.ShapeDtypeStruct((N,), jnp.float32),
        ],
        mesh=_sc_mesh(),
        scratch_types=[
            pltpu.VMEM((RPT_D, EW), jnp.int32),
            pltpu.VMEM((128,), jnp.float32),
            pltpu.VMEM((RPT,), jnp.float32),
            pltpu.VMEM((TPC,), jnp.float32),
            pltpu.VMEM((TPC, F), jnp.float32),
            pltpu.VMEM((80, F), jnp.float32),
            pltpu.SemaphoreType.DMA,
            [pltpu.SemaphoreType.DMA] * (TPC // 80),
            pltpu.VMEM_SHARED((NP,), jnp.float32),
        ],
    )


def _pre_body(ei_hbm, x_hbm, y_hbm, xs_hbm, dis_hbm,
              idx_v, ones_v, zero_v, disv, xbig, ybuf, dsem, xsems, deg_sh):
    c = lax.axis_index("c")
    s = lax.axis_index("s")

    base = c * HALF + s * TPC
    nch = jnp.minimum(TPC // 80, (N - base) // 80)
    for j in range(TPC // 80):
        @pl.when(j < nch)
        def _(j=j):
            pltpu.async_copy(x_hbm.at[pl.ds(base + j * 80, 80)],
                             xbig.at[pl.ds(j * 80, 80)], xsems[j])

    def _zero(i, _):
        zero_v[pl.ds(i * 16, 16)] = jnp.zeros((16,), jnp.float32)
        return 0

    lax.fori_loop(0, RPT // 16, _zero, 0)
    for j in range(8):
        ones_v[pl.ds(j * 16, 16)] = jnp.ones((16,), jnp.float32)
    pltpu.sync_copy(zero_v, deg_sh.at[pl.ds(s * RPT, RPT)])
    plsc.subcore_barrier()

    pltpu.sync_copy(ei_hbm.at[1, pl.ds(s * RPT_D, RPT_D)], idx_v)

    def _scat(t, _):
        pltpu.async_copy(ones_v.at[pl.ds(0, EW)],
                         deg_sh.at[idx_v.at[t]], dsem, add=True)

        @pl.when(t >= 8)
        def _():
            pltpu.make_async_copy(ones_v.at[pl.ds(0, EW)],
                                  deg_sh.at[idx_v.at[0]], dsem).wait()

        return 0

    lax.fori_loop(0, RPT_D, _scat, 0)

    def _drain(t, _):
        pltpu.make_async_copy(ones_v.at[pl.ds(0, EW)],
                              deg_sh.at[idx_v.at[0]], dsem).wait()
        return 0

    lax.fori_loop(0, 8, _drain, 0)
    plsc.subcore_barrier()

    pltpu.sync_copy(deg_sh.at[pl.ds(base, TPC)], disv)

    def _newton(i, _):
        d = disv[pl.ds(i * 16, 16)] + 1.0
        r = 1.0 / d
        for _i in range(24):
            r = r * (1.5 - 0.5 * d * r * r)
        disv[pl.ds(i * 16, 16)] = r
        return 0

    lax.fori_loop(0, TPC // 16, _newton, 0)

    for j in range(TPC // 80):
        @pl.when(j < nch)
        def _(j=j):
            row0 = base + j * 80
            pltpu.make_async_copy(x_hbm.at[pl.ds(row0, 80)],
                                  xbig.at[pl.ds(j * 80, 80)],
                                  xsems[j]).wait()

            def _grp(g, _2):
                dvec = disv[pl.ds(j * 80 + g * 16, 16)]
                for r16 in range(16):
                    r = j * 80 + g * 16 + r16
                    dsc = dvec.at[jnp.full((16,), r16, jnp.int32)].get(
                        mode="promise_in_bounds")
                    for l in range(F // 16):
                        sl = pl.ds(l * 16, 16)
                        yv = xbig[r, sl] * dsc
                        ybuf[g * 16 + r16, sl] = yv
                        xbig[r, sl] = yv * dsc
                return 0

            lax.fori_loop(0, 80 // 16, _grp, 0)
            pltpu.sync_copy(ybuf, y_hbm.at[pl.ds(row0, 80)])
            pltpu.sync_copy(xbig.at[pl.ds(j * 80, 80)],
                            xs_hbm.at[pl.ds(row0, 80)])
            pltpu.sync_copy(disv.at[pl.ds(j * 80, 80)],
                            dis_hbm.at[pl.ds(row0, 80)])


_CH = 40


@functools.cache
def _agg_call():
    return pl.kernel(
        _agg_body,
        out_type=jax.ShapeDtypeStruct((NC, NP, F), jnp.float32),
        mesh=_sc_mesh(),
        scratch_types=[
            pltpu.VMEM((_CH, EW), jnp.int32),
            pltpu.VMEM((_CH, EW), jnp.int32),
            pltpu.VMEM((EW, F), jnp.float32),
            pltpu.VMEM((EW, F), jnp.float32),
            pltpu.SemaphoreType.DMA,
            pltpu.SemaphoreType.DMA,
            pltpu.SemaphoreType.DMA,
            pltpu.SemaphoreType.DMA,
            pltpu.VMEM_SHARED((NP, F), jnp.float32),
        ],
    )


def _agg_body(ei_hbm, y_hbm, zp_hbm,
              idx_s, idx_d, buf0, buf1, g0, g1, s0, s1, z_sh):
    c = lax.axis_index("c")
    s = lax.axis_index("s")
    w = c * NS + s

    bufs = (buf0, buf1)
    gsem = (g0, g1)
    ssem = (s0, s1)

    def fire_g(t, b):
        pltpu.async_copy(y_hbm.at[idx_s.at[t]], bufs[b], gsem[b])

    def wait_g(b):
        pltpu.make_async_copy(y_hbm.at[idx_s.at[0]], bufs[b], gsem[b]).wait()

    def fire_s(t, b):
        pltpu.async_copy(bufs[b], z_sh.at[idx_d.at[t]], ssem[b], add=True)

    def wait_s(b):
        pltpu.make_async_copy(bufs[b], z_sh.at[idx_d.at[0]], ssem[b]).wait()

    def _zero(r, _):
        for j in range(F // 16):
            buf0[r, pl.ds(j * 16, 16)] = jnp.zeros((16,), jnp.float32)
        return 0

    lax.fori_loop(0, EW, _zero, 0)
    for j in range(RPT // 80):
        pltpu.sync_copy(buf0.at[pl.ds(0, 80)],
                        z_sh.at[pl.ds(s * RPT + j * 80, 80)])
    plsc.subcore_barrier()

    for chunk in range(RPW // _CH):
        pltpu.sync_copy(ei_hbm.at[0, pl.ds(w * RPW + chunk * _CH, _CH)], idx_s)
        pltpu.sync_copy(ei_hbm.at[1, pl.ds(w * RPW + chunk * _CH, _CH)], idx_d)

        fire_g(0, 0)

        def _pair(tp, _):
            t0 = 2 * tp
            fire_g(t0 + 1, 1)
            wait_g(0)
            pltpu.sync_copy(buf0, z_sh.at[idx_d.at[t0]], add=True)

            @pl.when(tp != _CH // 2 - 1)
            def _():
                fire_g(t0 + 2, 0)

            wait_g(1)
            pltpu.sync_copy(buf1, z_sh.at[idx_d.at[t0 + 1]], add=True)
            return 0

        lax.fori_loop(0, _CH // 2, _pair, 0)

    plsc.subcore_barrier()
    pltpu.sync_copy(z_sh.at[pl.ds(s * RPT, RPT)],
                    zp_hbm.at[c, pl.ds(s * RPT, RPT)])


def _mm_body(zp_ref, dis_ref, xs_ref, Ws_ref, bs_ref, out_ref):
    z = zp_ref[0] + zp_ref[1]
    zs = dis_ref[...] * z + xs_ref[...]
    for k in range(K):
        hk = lax.dot_general(zs, Ws_ref[k], (((1,), (1,)), ((), ())),
                             preferred_element_type=jnp.float32)
        out_ref[:, k, :] = hk + bs_ref[k]


_mm_call = pl.pallas_call(
    _mm_body,
    grid=(N // _B,),
    in_specs=[
        pl.BlockSpec((NC, _B, F), lambda i: (0, i, 0)),
        pl.BlockSpec((_B, 1), lambda i: (i, 0)),
        pl.BlockSpec((_B, F), lambda i: (i, 0)),
        pl.BlockSpec((K, F, F), lambda i: (0, 0, 0)),
        pl.BlockSpec((K, 1, F), lambda i: (0, 0, 0)),
    ],
    out_specs=pl.BlockSpec((_B, K, F), lambda i: (i, 0, 0)),
    out_shape=jax.ShapeDtypeStruct((N, K, F), jnp.float32),
)


def kernel(x, edge_index, Ws, bs):
    pad_src = jnp.arange(EPAD, dtype=jnp.int32) % N
    pad_dst = N + jnp.arange(EPAD, dtype=jnp.int32) % (NP - N)
    ei3 = jnp.concatenate(
        [edge_index, jnp.stack([pad_src, pad_dst])], axis=1).reshape(2, ER, EW)
    y, xs, dis = _pre_call()(ei3, x)
    zp = _agg_call()(ei3, y)
    return _mm_call(zp, dis.reshape(N, 1), xs, Ws, bs.reshape(K, 1, F))
